# baseline (speedup 1.0000x reference)
import functools
import numpy as np
import jax
import jax.numpy as jnp
from jax import lax
from jax.experimental import pallas as pl
from jax.experimental.pallas import tpu as pltpu, tpu_sc as plsc

N = 10000
G = 10
NPG = 1000
E = 160000
POOL = 100
QS = [2, 4, 8, 16, 32, 64]
QOFF = np.cumsum([0] + QS)
QTOT = int(QOFF[-1])

NP = 10240
EP = 163840
EB = 128
NBLK = EP // (32 * EB)
RPT = NP // 16


def _edge_segsum(table_pad, src2d, dst2d, coef2d=None):
    D = table_pad.shape[1]
    RING = 2 if D >= 128 else 6
    HALF = RING // 2
    scaled = coef2d is not None
    mesh = plsc.VectorSubcoreMesh(core_axis_name="c", subcore_axis_name="s")

    scratch = [
        pltpu.VMEM((NBLK, EB), jnp.int32),
        pltpu.VMEM((NBLK, EB), jnp.int32),
    ]
    scratch += [pltpu.VMEM((EB, D), jnp.float32) for _ in range(RING)]
    if scaled:
        scratch += [pltpu.VMEM((EB, D), jnp.float32) for _ in range(RING)]
        scratch += [pltpu.SemaphoreType.DMA for _ in range(RING)]
    scratch += [pltpu.VMEM_SHARED((NP, D), jnp.float32)]
    scratch += [pltpu.SemaphoreType.DMA for _ in range(2 * RING)]

    @functools.partial(
        pl.kernel,
        out_type=jax.ShapeDtypeStruct((2, NP, D), jnp.float32),
        mesh=mesh,
        scratch_types=scratch,
        compiler_params=(None if D >= 128 else
                         pltpu.CompilerParams(use_tc_tiling_on_sc=False)),
    )
    def k(*refs):
        if scaled:
            (table_hbm, src_hbm, dst_hbm, cexp_hbm, zeros_hbm, out_hbm,
             srcv, dstv, *rest) = refs
            bufs = rest[:RING]
            cbufs = rest[RING:2 * RING]
            csems = rest[2 * RING:3 * RING]
            rest = rest[3 * RING:]
        else:
            (table_hbm, src_hbm, dst_hbm, zeros_hbm, out_hbm,
             srcv, dstv, *rest) = refs
            cexp_hbm = None
            bufs = rest[:RING]
            rest = rest[RING:]
        accum = rest[0]
        gsems = rest[1:1 + RING]
        ssems = rest[1 + RING:]
        c = lax.axis_index("c")
        s = lax.axis_index("s")
        tile = c * 16 + s
        pltpu.sync_copy(src_hbm.at[pl.ds(tile * NBLK, NBLK)], srcv)
        pltpu.sync_copy(dst_hbm.at[pl.ds(tile * NBLK, NBLK)], dstv)
        pltpu.sync_copy(zeros_hbm.at[pl.ds(s * RPT, RPT)],
                        accum.at[pl.ds(s * RPT, RPT)])
        plsc.subcore_barrier()

        def scale_block(buf, cbuf):
            def body(e, _):
                for k2 in range(D // 16):
                    buf[e, pl.ds(k2 * 16, 16)] = (
                        buf[e, pl.ds(k2 * 16, 16)]
                        * cbuf[e, pl.ds(k2 * 16, 16)])
                return 0

            lax.fori_loop(0, EB, body, 0)

        cd = [None] * NBLK
        gd = [None] * NBLK
        def start_block(j):
            b = j % RING
            gd[j] = pltpu.async_copy(
                table_hbm.at[srcv.at[j]], bufs[b], gsems[b])
            if scaled:
                cd[j] = pltpu.async_copy(
                    cexp_hbm.at[pl.ds((tile * NBLK + j) * EB, EB)],
                    cbufs[b], csems[b])

        sd = [None] * NBLK
        for j in range(HALF):
            start_block(j)
        for j in range(NBLK):
            jn = j + HALF
            if jn < NBLK:
                if jn - RING >= 0:
                    sd[jn - RING].wait()
                start_block(jn)
            b = j % RING
            gd[j].wait()
            if scaled:
                cd[j].wait()
                scale_block(bufs[b], cbufs[b])
            sd[j] = pltpu.async_copy(
                bufs[b], accum.at[dstv.at[j]], ssems[b], add=True)
        for j in range(max(0, NBLK - RING + HALF - HALF), NBLK):
            if j >= NBLK - RING:
                sd[j].wait()
        plsc.subcore_barrier()
        pltpu.sync_copy(accum.at[pl.ds(s * RPT, RPT)],
                        out_hbm.at[c, pl.ds(s * RPT, RPT)])

    zeros_pad = jnp.zeros((NP, D), jnp.float32)
    if scaled:
        return k(table_pad, src2d, dst2d, coef2d, zeros_pad)
    return k(table_pad, src2d, dst2d, zeros_pad)


def _bp_softmax_body(p0_ref, p1_ref, beta_ref, out_ref):
    m = (p0_ref[...] + p1_ref[...]) * beta_ref[...]
    parts = []
    for i, q in enumerate(QS):
        sub = m[:, QOFF[i]:QOFF[i + 1]]
        mx = jnp.max(sub, axis=-1, keepdims=True)
        e = jnp.exp(sub - mx)
        parts.append(e / jnp.sum(e, axis=-1, keepdims=True))
    parts.append(jnp.zeros((m.shape[0], 128 - QTOT), jnp.float32))
    out_ref[...] = jnp.concatenate(parts, axis=-1)


def _bp_softmax(part0, part1, beta_row):
    blk = 1280
    return pl.pallas_call(
        _bp_softmax_body,
        grid=(NP // blk,),
        in_specs=[
            pl.BlockSpec((blk, 128), lambda i: (i, 0)),
            pl.BlockSpec((blk, 128), lambda i: (i, 0)),
            pl.BlockSpec((1, 128), lambda i: (0, 0)),
        ],
        out_specs=pl.BlockSpec((blk, 128), lambda i: (i, 0)),
        out_shape=jax.ShapeDtypeStruct((NP, 128), jnp.float32),
    )(part0, part1, beta_row)


def _pad_edges(src, dst):
    npad = EP - E
    src_p = jnp.concatenate([src, jnp.full((npad,), N, jnp.int32)])
    dst_p = jnp.concatenate([dst, jnp.full((npad,), N, jnp.int32)])
    return src_p.reshape(EP // EB, EB), dst_p.reshape(EP // EB, EB)


def _bn(x, g, b):
    m = x.mean(0)
    v = x.var(0)
    return (x - m) / jnp.sqrt(v + 1e-5) * g + b


def _final_mlp_body(conv_ref, w1_ref, b1_ref, w2_ref, b2_ref, out_ref):
    h = jnp.maximum(
        jnp.dot(conv_ref[...], w1_ref[...], preferred_element_type=jnp.float32)
        + b1_ref[...], 0.0)
    out_ref[...] = (
        jnp.dot(h, w2_ref[...], preferred_element_type=jnp.float32) + b2_ref[...])


def _final_mlp(conv_out, w1, b1, w2, b2):
    return pl.pallas_call(
        _final_mlp_body,
        out_shape=jax.ShapeDtypeStruct((G, w2.shape[1]), jnp.float32),
    )(conv_out, w1, b1[None, :], w2, b2[None, :])


def kernel(x, edge_index, edge_attr, params):
    p = params
    src, dst = edge_index[0], edge_index[1]
    w = edge_attr
    order = jnp.argsort(dst)
    src = src[order]
    dst = dst[order]
    w = w[order]

    src2d, dst2d = _pad_edges(src, dst)
    npad = EP - E
    coef_pad = lambda v: jnp.concatenate(
        [v, jnp.zeros((npad,), jnp.float32)]).reshape(EP // EB, EB)

    w_padded = jnp.concatenate([w, jnp.zeros((npad,), jnp.float32)])
    w2d = jnp.broadcast_to(w_padded[:, None], (EP, 32)) + jnp.zeros(
        (EP, 32), jnp.float32)

    ones_pad = jnp.zeros((NP, 32), jnp.float32).at[:N, :].set(1.0)
    dparts = _edge_segsum(ones_pad, src2d, dst2d, w2d)
    deg = (dparts[0] + dparts[1])[:N, 0] + 1.0
    dinv = 1.0 / jnp.sqrt(deg)
    dinv2_self = dinv * dinv

    def gcn1(h):
        ht = h * dinv[:, None]
        h_pad = jnp.zeros((NP, 32), jnp.float32).at[:N, :30].set(ht)
        parts = _edge_segsum(h_pad, src2d, dst2d, w2d)
        agg = dinv[:, None] * (parts[0] + parts[1])[:N, :30]
        return agg + h * dinv2_self[:, None]

    x11 = _bn(gcn1(x @ p['W11']) + p['b11'], p['g11'], p['be11'])
    x12 = _bn(gcn1(x11 @ p['W12']) + p['b12'], p['g12'], p['be12'])
    x13 = _bn(gcn1(x12 @ p['W13']) + p['b13'], p['g13'], p['be13'])
    x1 = jnp.concatenate([x11, x12, x13], axis=-1)
    x1_out = x1.reshape(G, NPG, -1).max(axis=1)

    psis = [jax.nn.softmax(
        jax.random.normal(jax.random.key(100 + i), (N, q)), axis=-1)
        for i, q in enumerate(QS)]
    psi0 = jnp.concatenate(psis, axis=-1)
    psi_pad = jnp.zeros((NP, 128), jnp.float32).at[:N, :QTOT].set(psi0)
    beta_row = jnp.zeros((1, 128), jnp.float32)
    for i in range(len(QS)):
        beta_row = beta_row.at[0, QOFF[i]:QOFF[i + 1]].set(p['beta'][i])
    for _ in range(10):
        parts01 = _edge_segsum(psi_pad, src2d, dst2d)
        psi_pad = _bp_softmax(parts01[0], parts01[1], beta_row)
    psi_all = psi_pad[:N, :QTOT]

    s1 = psi_all @ p['Wp'] + p['bpb']
    s = jax.nn.softmax(s1.reshape(G, NPG, POOL), axis=-1)
    x13r = x13.reshape(G, NPG, 30)
    xp = jnp.einsum('gnk,gnd->gkd', s, x13r)

    s_flat = s.reshape(N, POOL)
    seg2 = (src // NPG) * NPG + (dst % NPG)
    seg2d = jnp.concatenate(
        [seg2, jnp.full((EP - E,), N, jnp.int32)]).reshape(EP // EB, EB)
    t_chunks = []
    for cb in range(0, POOL, 25):
        s_pad = jnp.zeros((NP, 32), jnp.float32).at[:N, :25].set(
            s_flat[:, cb:cb + 25])
        tparts = _edge_segsum(s_pad, src2d, seg2d, w2d)
        t_chunks.append((tparts[0] + tparts[1])[:N, :25])
    t_seg = jnp.concatenate(t_chunks, axis=-1).reshape(G, NPG, POOL)
    adjp = jnp.einsum('gmk,gml->gkl', t_seg, s)

    deg2 = adjp.sum(axis=1) + 1.0
    dinv2 = 1.0 / jnp.sqrt(deg2)

    def gcn2(h, W, b):
        hh = h @ W
        hs = hh * dinv2[:, :, None]
        agg = jnp.einsum('gij,gid->gjd', adjp, hs)
        return (agg * dinv2[:, :, None] + hh * (dinv2 ** 2)[:, :, None] + b)

    x21 = _bn(gcn2(xp, p['W21'], p['b21']).reshape(G * POOL, 30),
              p['g21'], p['be21'])
    x22 = _bn(gcn2(x21.reshape(G, POOL, 30), p['W22'], p['b22']).reshape(G * POOL, 30),
              p['g22'], p['be22'])
    x23 = _bn(gcn2(x22.reshape(G, POOL, 30), p['W23'], p['b23']).reshape(G * POOL, 30),
              p['g23'], p['be23'])
    x2 = jnp.concatenate([x21, x22, x23], axis=-1)
    x2_out = x2.reshape(G, POOL, -1).max(axis=1)
    conv_out = jnp.concatenate([x1_out, x2_out], axis=-1)

    out = _final_mlp(conv_out, p['Wf1'], p['bf1'], p['Wf2'], p['bf2'])
    return (out, jnp.zeros((1,), jnp.float32))


# speedup vs baseline: 1.0658x; 1.0658x over previous
import functools
import numpy as np
import jax
import jax.numpy as jnp
from jax import lax
from jax.experimental import pallas as pl
from jax.experimental.pallas import tpu as pltpu, tpu_sc as plsc

N = 10000
G = 10
NPG = 1000
E = 160000
POOL = 100
QS = [2, 4, 8, 16, 32, 64]
QOFF = np.cumsum([0] + QS)
QTOT = int(QOFF[-1])

NP = 10240
EP = 163840
EB = 128
NBLK = EP // (32 * EB)
RPT = NP // 16


def _edge_segsum(table_pad, src2d, dst2d, coef2d=None):
    D = table_pad.shape[1]
    RING = 2 if D >= 128 else 6
    HALF = RING // 2
    scaled = coef2d is not None
    mesh = plsc.VectorSubcoreMesh(core_axis_name="c", subcore_axis_name="s")

    scratch = [
        pltpu.VMEM((NBLK, EB), jnp.int32),
        pltpu.VMEM((NBLK, EB), jnp.int32),
    ]
    scratch += [pltpu.VMEM((EB, D), jnp.float32) for _ in range(RING)]
    if scaled:
        scratch += [pltpu.VMEM((EB, D), jnp.float32) for _ in range(RING)]
        scratch += [pltpu.SemaphoreType.DMA for _ in range(RING)]
    scratch += [pltpu.VMEM_SHARED((NP, D), jnp.float32)]
    scratch += [pltpu.SemaphoreType.DMA for _ in range(2 * RING)]

    @functools.partial(
        pl.kernel,
        out_type=jax.ShapeDtypeStruct((2, NP, D), jnp.float32),
        mesh=mesh,
        scratch_types=scratch,
        compiler_params=(None if D >= 128 else
                         pltpu.CompilerParams(use_tc_tiling_on_sc=False)),
    )
    def k(*refs):
        if scaled:
            (table_hbm, src_hbm, dst_hbm, cexp_hbm, zeros_hbm, out_hbm,
             srcv, dstv, *rest) = refs
            bufs = rest[:RING]
            cbufs = rest[RING:2 * RING]
            csems = rest[2 * RING:3 * RING]
            rest = rest[3 * RING:]
        else:
            (table_hbm, src_hbm, dst_hbm, zeros_hbm, out_hbm,
             srcv, dstv, *rest) = refs
            cexp_hbm = None
            bufs = rest[:RING]
            rest = rest[RING:]
        accum = rest[0]
        gsems = rest[1:1 + RING]
        ssems = rest[1 + RING:]
        c = lax.axis_index("c")
        s = lax.axis_index("s")
        tile = c * 16 + s
        pltpu.sync_copy(src_hbm.at[pl.ds(tile * NBLK, NBLK)], srcv)
        pltpu.sync_copy(dst_hbm.at[pl.ds(tile * NBLK, NBLK)], dstv)
        pltpu.sync_copy(zeros_hbm.at[pl.ds(s * RPT, RPT)],
                        accum.at[pl.ds(s * RPT, RPT)])
        plsc.subcore_barrier()

        def scale_block(buf, cbuf):
            def body(e, _):
                for k2 in range(D // 16):
                    buf[e, pl.ds(k2 * 16, 16)] = (
                        buf[e, pl.ds(k2 * 16, 16)]
                        * cbuf[e, pl.ds(k2 * 16, 16)])
                return 0

            lax.fori_loop(0, EB, body, 0)

        cd = [None] * NBLK
        gd = [None] * NBLK
        def start_block(j):
            b = j % RING
            gd[j] = pltpu.async_copy(
                table_hbm.at[srcv.at[j]], bufs[b], gsems[b])
            if scaled:
                cd[j] = pltpu.async_copy(
                    cexp_hbm.at[pl.ds((tile * NBLK + j) * EB, EB)],
                    cbufs[b], csems[b])

        sd = [None] * NBLK
        for j in range(HALF):
            start_block(j)
        for j in range(NBLK):
            jn = j + HALF
            if jn < NBLK:
                if jn - RING >= 0:
                    sd[jn - RING].wait()
                start_block(jn)
            b = j % RING
            gd[j].wait()
            if scaled:
                cd[j].wait()
                scale_block(bufs[b], cbufs[b])
            sd[j] = pltpu.async_copy(
                bufs[b], accum.at[dstv.at[j]], ssems[b], add=True)
        for j in range(max(0, NBLK - RING + HALF - HALF), NBLK):
            if j >= NBLK - RING:
                sd[j].wait()
        plsc.subcore_barrier()
        pltpu.sync_copy(accum.at[pl.ds(s * RPT, RPT)],
                        out_hbm.at[c, pl.ds(s * RPT, RPT)])

    zeros_pad = jnp.zeros((NP, D), jnp.float32)
    if scaled:
        return k(table_pad, src2d, dst2d, coef2d, zeros_pad)
    return k(table_pad, src2d, dst2d, zeros_pad)


def _bp_softmax_body(p0_ref, p1_ref, beta_ref, out_ref):
    m = (p0_ref[...] + p1_ref[...]) * beta_ref[...]
    parts = []
    for i, q in enumerate(QS):
        sub = m[:, QOFF[i]:QOFF[i + 1]]
        mx = jnp.max(sub, axis=-1, keepdims=True)
        e = jnp.exp(sub - mx)
        parts.append(e / jnp.sum(e, axis=-1, keepdims=True))
    parts.append(jnp.zeros((m.shape[0], 128 - QTOT), jnp.float32))
    out_ref[...] = jnp.concatenate(parts, axis=-1)


def _bp_softmax(part0, part1, beta_row):
    blk = 1280
    return pl.pallas_call(
        _bp_softmax_body,
        grid=(NP // blk,),
        in_specs=[
            pl.BlockSpec((blk, 128), lambda i: (i, 0)),
            pl.BlockSpec((blk, 128), lambda i: (i, 0)),
            pl.BlockSpec((1, 128), lambda i: (0, 0)),
        ],
        out_specs=pl.BlockSpec((blk, 128), lambda i: (i, 0)),
        out_shape=jax.ShapeDtypeStruct((NP, 128), jnp.float32),
    )(part0, part1, beta_row)


def _pad_edges(src, dst):
    npad = EP - E
    src_p = jnp.concatenate([src, jnp.full((npad,), N, jnp.int32)])
    dst_p = jnp.concatenate([dst, jnp.full((npad,), N, jnp.int32)])
    return src_p.reshape(EP // EB, EB), dst_p.reshape(EP // EB, EB)


def _bn(x, g, b):
    m = x.mean(0)
    v = x.var(0)
    return (x - m) / jnp.sqrt(v + 1e-5) * g + b


def _dinv_body(p0_ref, p1_ref, out_ref):
    deg = p0_ref[:N, :1] + p1_ref[:N, :1] + 1.0
    out_ref[...] = 1.0 / jnp.sqrt(deg)


def _dinv_kernel(dparts):
    return pl.pallas_call(
        _dinv_body,
        out_shape=jax.ShapeDtypeStruct((N, 1), jnp.float32),
    )(dparts[0], dparts[1])


def _gcn_pre_body(xp_ref, w_ref, dinv_ref, h_ref, htp_ref):
    h = jnp.dot(xp_ref[...], w_ref[...], preferred_element_type=jnp.float32)
    h_ref[...] = h
    ht = h * dinv_ref[...]
    ht32 = jnp.concatenate([ht, jnp.zeros((N, 2), jnp.float32)], axis=1)
    htp_ref[...] = jnp.concatenate(
        [ht32, jnp.zeros((NP - N, 32), jnp.float32)], axis=0)


def _gcn_pre(xin, W, dinv):
    din = xin.shape[1]
    return pl.pallas_call(
        _gcn_pre_body,
        out_shape=(jax.ShapeDtypeStruct((N, 30), jnp.float32),
                   jax.ShapeDtypeStruct((NP, 32), jnp.float32)),
    )(xin, W, dinv)


def _gcn_post_body(p0_ref, p1_ref, h_ref, dinv_ref, b_ref, g_ref, be_ref,
                   out_ref):
    dinv = dinv_ref[...]
    agg = dinv * (p0_ref[:N, :30] + p1_ref[:N, :30])
    pre = agg + h_ref[...] * (dinv * dinv) + b_ref[...]
    out_ref[...] = _bn(pre, g_ref[...], be_ref[...])


def _gcn_post(parts, h, dinv, b, g, be):
    return pl.pallas_call(
        _gcn_post_body,
        out_shape=jax.ShapeDtypeStruct((N, 30), jnp.float32),
    )(parts[0], parts[1], h, dinv, b[None, :], g[None, :], be[None, :])


def _assign_body(psi_ref, wp_ref, bpb_ref, x13_ref, s_ref, xp_ref):
    s1 = jnp.dot(psi_ref[:N, :QTOT], wp_ref[...],
                 preferred_element_type=jnp.float32) + bpb_ref[...]
    mx = jnp.max(s1, axis=-1, keepdims=True)
    e = jnp.exp(s1 - mx)
    s_flat = e / jnp.sum(e, axis=-1, keepdims=True)
    s_ref[...] = s_flat
    s3 = s_flat.reshape(G, NPG, POOL)
    x13r = x13_ref[...].reshape(G, NPG, 30)
    xp_ref[...] = jnp.einsum('gnk,gnd->gkd', s3, x13r,
                             preferred_element_type=jnp.float32)


def _assign(psi_pad, Wp, bpb, x13):
    return pl.pallas_call(
        _assign_body,
        out_shape=(jax.ShapeDtypeStruct((N, POOL), jnp.float32),
                   jax.ShapeDtypeStruct((G, POOL, 30), jnp.float32)),
    )(psi_pad, Wp, bpb[None, :], x13)


def _pooled_body(tp_refs, s_ref, xp_ref, x1c_ref, prm_refs, out_ref):
    (w21, b21, g21, be21, w22, b22, g22, be22,
     w23, b23, g23, be23, wf1, bf1, wf2, bf2) = prm_refs
    t_cols = []
    for c in range(4):
        p0, p1 = tp_refs[2 * c], tp_refs[2 * c + 1]
        t_cols.append(p0[:N, :25] + p1[:N, :25])
    t_seg = jnp.concatenate(t_cols, axis=-1).reshape(G, NPG, POOL)
    s3 = s_ref[...].reshape(G, NPG, POOL)
    adjp = jnp.einsum('gmk,gml->gkl', t_seg, s3,
                      preferred_element_type=jnp.float32)
    deg2 = adjp.sum(axis=1) + 1.0
    dinv2 = 1.0 / jnp.sqrt(deg2)

    def gcn2(h, W, b):
        hh = jnp.einsum('gnd,de->gne', h, W[...],
                        preferred_element_type=jnp.float32)
        hs = hh * dinv2[:, :, None]
        agg = jnp.einsum('gij,gid->gjd', adjp, hs,
                         preferred_element_type=jnp.float32)
        return (agg * dinv2[:, :, None]
                + hh * (dinv2 ** 2)[:, :, None] + b[...])

    x21 = _bn(gcn2(xp_ref[...], w21, b21).reshape(G * POOL, 30),
              g21[...], be21[...])
    x22 = _bn(gcn2(x21.reshape(G, POOL, 30), w22, b22).reshape(G * POOL, 30),
              g22[...], be22[...])
    x23 = _bn(gcn2(x22.reshape(G, POOL, 30), w23, b23).reshape(G * POOL, 30),
              g23[...], be23[...])
    x2 = jnp.concatenate([x21, x22, x23], axis=-1)
    x2_out = x2.reshape(G, POOL, 90).max(axis=1)
    x1_out = x1c_ref[...].reshape(G, NPG, 90).max(axis=1)
    conv_out = jnp.concatenate([x1_out, x2_out], axis=-1)
    hmid = jnp.maximum(
        jnp.dot(conv_out, wf1[...], preferred_element_type=jnp.float32)
        + bf1[...], 0.0)
    out_ref[...] = (jnp.dot(hmid, wf2[...],
                            preferred_element_type=jnp.float32) + bf2[...])


def _pooled(tparts_list, s_flat, xp, x1c, p):
    nt = len(tparts_list)

    def body(*refs):
        tp_refs = refs[:2 * nt]
        s_ref, xp_ref, x1c_ref = refs[2 * nt:2 * nt + 3]
        prm_refs = refs[2 * nt + 3:-1]
        _pooled_body(tp_refs, s_ref, xp_ref, x1c_ref, prm_refs, refs[-1])

    args = []
    for tp in tparts_list:
        args += [tp[0], tp[1]]
    args += [s_flat, xp, x1c]
    args += [p['W21'], p['b21'][None, :], p['g21'][None, :], p['be21'][None, :],
             p['W22'], p['b22'][None, :], p['g22'][None, :], p['be22'][None, :],
             p['W23'], p['b23'][None, :], p['g23'][None, :], p['be23'][None, :],
             p['Wf1'], p['bf1'][None, :], p['Wf2'], p['bf2'][None, :]]
    return pl.pallas_call(
        body,
        out_shape=jax.ShapeDtypeStruct((G, 6), jnp.float32),
    )(*args)


def kernel(x, edge_index, edge_attr, params):
    p = params
    src, dst = edge_index[0], edge_index[1]
    w = edge_attr

    src2d, dst2d = _pad_edges(src, dst)
    npad = EP - E
    coef_pad = lambda v: jnp.concatenate(
        [v, jnp.zeros((npad,), jnp.float32)]).reshape(EP // EB, EB)

    w_padded = jnp.concatenate([w, jnp.zeros((npad,), jnp.float32)])
    w2d = jnp.broadcast_to(w_padded[:, None], (EP, 32)) + jnp.zeros(
        (EP, 32), jnp.float32)

    ones_pad = jnp.zeros((NP, 32), jnp.float32).at[:N, :].set(1.0)
    dparts = _edge_segsum(ones_pad, src2d, dst2d, w2d)
    dinv = _dinv_kernel(dparts)

    def gcn1(xin, W, b, g, be):
        h, h_pad = _gcn_pre(xin, W, dinv)
        parts = _edge_segsum(h_pad, src2d, dst2d, w2d)
        return _gcn_post(parts, h, dinv, b, g, be)

    x11 = gcn1(x, p['W11'], p['b11'], p['g11'], p['be11'])
    x12 = gcn1(x11, p['W12'], p['b12'], p['g12'], p['be12'])
    x13 = gcn1(x12, p['W13'], p['b13'], p['g13'], p['be13'])
    x1c = jnp.concatenate([x11, x12, x13], axis=-1)

    psis = [jax.nn.softmax(
        jax.random.normal(jax.random.key(100 + i), (N, q)), axis=-1)
        for i, q in enumerate(QS)]
    psi0 = jnp.concatenate(psis, axis=-1)
    psi_pad = jnp.zeros((NP, 128), jnp.float32).at[:N, :QTOT].set(psi0)
    beta_row = jnp.zeros((1, 128), jnp.float32)
    for i in range(len(QS)):
        beta_row = beta_row.at[0, QOFF[i]:QOFF[i + 1]].set(p['beta'][i])
    for _ in range(10):
        parts01 = _edge_segsum(psi_pad, src2d, dst2d)
        psi_pad = _bp_softmax(parts01[0], parts01[1], beta_row)

    s_flat, xp = _assign(psi_pad, p['Wp'], p['bpb'], x13)

    seg2 = (src // NPG) * NPG + (dst % NPG)
    seg2d = jnp.concatenate(
        [seg2, jnp.full((EP - E,), N, jnp.int32)]).reshape(EP // EB, EB)
    tparts_list = []
    for cb in range(0, POOL, 25):
        s_pad = jnp.zeros((NP, 32), jnp.float32).at[:N, :25].set(
            s_flat[:, cb:cb + 25])
        tparts_list.append(_edge_segsum(s_pad, src2d, seg2d, w2d))

    out = _pooled(tparts_list, s_flat, xp, x1c, p)
    return (out, jnp.zeros((1,), jnp.float32))


# speedup vs baseline: 1.0769x; 1.0104x over previous
import functools
import numpy as np
import jax
import jax.numpy as jnp
from jax import lax
from jax.experimental import pallas as pl
from jax.experimental.pallas import tpu as pltpu, tpu_sc as plsc

N = 10000
G = 10
NPG = 1000
E = 160000
POOL = 100
QS = [2, 4, 8, 16, 32, 64]
QOFF = np.cumsum([0] + QS)
QTOT = int(QOFF[-1])

NP = 10240
EP = 163840
EB = 128
NBLK = EP // (32 * EB)
RPT = NP // 16


def _edge_segsum(table_pad, src2d, dst2d, coef2d=None):
    D = table_pad.shape[1]
    RING = 2 if D >= 128 else (4 if D == 64 else 6)
    HALF = RING // 2
    scaled = coef2d is not None
    mesh = plsc.VectorSubcoreMesh(core_axis_name="c", subcore_axis_name="s")

    scratch = [
        pltpu.VMEM((NBLK, EB), jnp.int32),
        pltpu.VMEM((NBLK, EB), jnp.int32),
    ]
    scratch += [pltpu.VMEM((EB, D), jnp.float32) for _ in range(RING)]
    if scaled:
        scratch += [pltpu.VMEM((EB, D), jnp.float32) for _ in range(RING)]
        scratch += [pltpu.SemaphoreType.DMA for _ in range(RING)]
    scratch += [pltpu.VMEM_SHARED((NP, D), jnp.float32)]
    scratch += [pltpu.SemaphoreType.DMA for _ in range(2 * RING)]

    @functools.partial(
        pl.kernel,
        out_type=jax.ShapeDtypeStruct((2, NP, D), jnp.float32),
        mesh=mesh,
        scratch_types=scratch,
        compiler_params=(None if D >= 128 else
                         pltpu.CompilerParams(use_tc_tiling_on_sc=False)),
    )
    def k(*refs):
        if scaled:
            (table_hbm, src_hbm, dst_hbm, cexp_hbm, zeros_hbm, out_hbm,
             srcv, dstv, *rest) = refs
            bufs = rest[:RING]
            cbufs = rest[RING:2 * RING]
            csems = rest[2 * RING:3 * RING]
            rest = rest[3 * RING:]
        else:
            (table_hbm, src_hbm, dst_hbm, zeros_hbm, out_hbm,
             srcv, dstv, *rest) = refs
            cexp_hbm = None
            bufs = rest[:RING]
            rest = rest[RING:]
        accum = rest[0]
        gsems = rest[1:1 + RING]
        ssems = rest[1 + RING:]
        c = lax.axis_index("c")
        s = lax.axis_index("s")
        tile = c * 16 + s
        pltpu.sync_copy(src_hbm.at[pl.ds(tile * NBLK, NBLK)], srcv)
        pltpu.sync_copy(dst_hbm.at[pl.ds(tile * NBLK, NBLK)], dstv)
        pltpu.sync_copy(zeros_hbm.at[pl.ds(s * RPT, RPT)],
                        accum.at[pl.ds(s * RPT, RPT)])
        plsc.subcore_barrier()

        def scale_block(buf, cbuf):
            def body(e, _):
                for k2 in range(D // 16):
                    buf[e, pl.ds(k2 * 16, 16)] = (
                        buf[e, pl.ds(k2 * 16, 16)]
                        * cbuf[e, pl.ds(k2 * 16, 16)])
                return 0

            lax.fori_loop(0, EB, body, 0)

        cd = [None] * NBLK
        gd = [None] * NBLK
        def start_block(j):
            b = j % RING
            gd[j] = pltpu.async_copy(
                table_hbm.at[srcv.at[j]], bufs[b], gsems[b])
            if scaled:
                cd[j] = pltpu.async_copy(
                    cexp_hbm.at[pl.ds((tile * NBLK + j) * EB, EB)],
                    cbufs[b], csems[b])

        sd = [None] * NBLK
        for j in range(HALF):
            start_block(j)
        for j in range(NBLK):
            jn = j + HALF
            if jn < NBLK:
                if jn - RING >= 0:
                    sd[jn - RING].wait()
                start_block(jn)
            b = j % RING
            gd[j].wait()
            if scaled:
                cd[j].wait()
                scale_block(bufs[b], cbufs[b])
            sd[j] = pltpu.async_copy(
                bufs[b], accum.at[dstv.at[j]], ssems[b], add=True)
        for j in range(max(0, NBLK - RING + HALF - HALF), NBLK):
            if j >= NBLK - RING:
                sd[j].wait()
        plsc.subcore_barrier()
        pltpu.sync_copy(accum.at[pl.ds(s * RPT, RPT)],
                        out_hbm.at[c, pl.ds(s * RPT, RPT)])

    zeros_pad = jnp.zeros((NP, D), jnp.float32)
    if scaled:
        return k(table_pad, src2d, dst2d, coef2d, zeros_pad)
    return k(table_pad, src2d, dst2d, zeros_pad)


def _bp_softmax_body(p0_ref, p1_ref, beta_ref, out_ref):
    m = (p0_ref[...] + p1_ref[...]) * beta_ref[...]
    parts = []
    for i, q in enumerate(QS):
        sub = m[:, QOFF[i]:QOFF[i + 1]]
        mx = jnp.max(sub, axis=-1, keepdims=True)
        e = jnp.exp(sub - mx)
        parts.append(e / jnp.sum(e, axis=-1, keepdims=True))
    parts.append(jnp.zeros((m.shape[0], 128 - QTOT), jnp.float32))
    out_ref[...] = jnp.concatenate(parts, axis=-1)


def _bp_softmax(part0, part1, beta_row):
    blk = 1280
    return pl.pallas_call(
        _bp_softmax_body,
        grid=(NP // blk,),
        in_specs=[
            pl.BlockSpec((blk, 128), lambda i: (i, 0)),
            pl.BlockSpec((blk, 128), lambda i: (i, 0)),
            pl.BlockSpec((1, 128), lambda i: (0, 0)),
        ],
        out_specs=pl.BlockSpec((blk, 128), lambda i: (i, 0)),
        out_shape=jax.ShapeDtypeStruct((NP, 128), jnp.float32),
    )(part0, part1, beta_row)


def _pad_edges(src, dst):
    npad = EP - E
    src_p = jnp.concatenate([src, jnp.full((npad,), N, jnp.int32)])
    dst_p = jnp.concatenate([dst, jnp.full((npad,), N, jnp.int32)])
    return src_p.reshape(EP // EB, EB), dst_p.reshape(EP // EB, EB)


def _bn(x, g, b):
    m = x.mean(0)
    v = x.var(0)
    return (x - m) / jnp.sqrt(v + 1e-5) * g + b


def _dinv_body(p0_ref, p1_ref, out_ref):
    deg = p0_ref[:N, :1] + p1_ref[:N, :1] + 1.0
    out_ref[...] = 1.0 / jnp.sqrt(deg)


def _dinv_kernel(dparts):
    return pl.pallas_call(
        _dinv_body,
        out_shape=jax.ShapeDtypeStruct((N, 1), jnp.float32),
    )(dparts[0], dparts[1])


def _gcn_pre_body(xp_ref, w_ref, dinv_ref, h_ref, htp_ref):
    h = jnp.dot(xp_ref[...], w_ref[...], preferred_element_type=jnp.float32)
    h_ref[...] = h
    ht = h * dinv_ref[...]
    ht32 = jnp.concatenate([ht, jnp.zeros((N, 2), jnp.float32)], axis=1)
    htp_ref[...] = jnp.concatenate(
        [ht32, jnp.zeros((NP - N, 32), jnp.float32)], axis=0)


def _gcn_pre(xin, W, dinv):
    din = xin.shape[1]
    return pl.pallas_call(
        _gcn_pre_body,
        out_shape=(jax.ShapeDtypeStruct((N, 30), jnp.float32),
                   jax.ShapeDtypeStruct((NP, 32), jnp.float32)),
    )(xin, W, dinv)


def _gcn_post_body(p0_ref, p1_ref, h_ref, dinv_ref, b_ref, g_ref, be_ref,
                   out_ref):
    dinv = dinv_ref[...]
    agg = dinv * (p0_ref[:N, :30] + p1_ref[:N, :30])
    pre = agg + h_ref[...] * (dinv * dinv) + b_ref[...]
    out_ref[...] = _bn(pre, g_ref[...], be_ref[...])


def _gcn_post(parts, h, dinv, b, g, be):
    return pl.pallas_call(
        _gcn_post_body,
        out_shape=jax.ShapeDtypeStruct((N, 30), jnp.float32),
    )(parts[0], parts[1], h, dinv, b[None, :], g[None, :], be[None, :])


def _assign_body(psi_ref, wp_ref, bpb_ref, x13_ref, s_ref, xp_ref):
    s1 = jnp.dot(psi_ref[:N, :QTOT], wp_ref[...],
                 preferred_element_type=jnp.float32) + bpb_ref[...]
    mx = jnp.max(s1, axis=-1, keepdims=True)
    e = jnp.exp(s1 - mx)
    s_flat = e / jnp.sum(e, axis=-1, keepdims=True)
    s_ref[...] = s_flat
    s3 = s_flat.reshape(G, NPG, POOL)
    x13r = x13_ref[...].reshape(G, NPG, 30)
    xp_ref[...] = jnp.einsum('gnk,gnd->gkd', s3, x13r,
                             preferred_element_type=jnp.float32)


def _assign(psi_pad, Wp, bpb, x13):
    return pl.pallas_call(
        _assign_body,
        out_shape=(jax.ShapeDtypeStruct((N, POOL), jnp.float32),
                   jax.ShapeDtypeStruct((G, POOL, 30), jnp.float32)),
    )(psi_pad, Wp, bpb[None, :], x13)


def _pooled_body(tp_refs, s_ref, xp_ref, x1c_ref, prm_refs, out_ref):
    (w21, b21, g21, be21, w22, b22, g22, be22,
     w23, b23, g23, be23, wf1, bf1, wf2, bf2) = prm_refs
    t_cols = []
    for c in range(2):
        p0, p1 = tp_refs[2 * c], tp_refs[2 * c + 1]
        t_cols.append(p0[:N, :50] + p1[:N, :50])
    t_seg = jnp.concatenate(t_cols, axis=-1).reshape(G, NPG, POOL)
    s3 = s_ref[...].reshape(G, NPG, POOL)
    adjp = jnp.einsum('gmk,gml->gkl', t_seg, s3,
                      preferred_element_type=jnp.float32)
    deg2 = adjp.sum(axis=1) + 1.0
    dinv2 = 1.0 / jnp.sqrt(deg2)

    def gcn2(h, W, b):
        hh = jnp.einsum('gnd,de->gne', h, W[...],
                        preferred_element_type=jnp.float32)
        hs = hh * dinv2[:, :, None]
        agg = jnp.einsum('gij,gid->gjd', adjp, hs,
                         preferred_element_type=jnp.float32)
        return (agg * dinv2[:, :, None]
                + hh * (dinv2 ** 2)[:, :, None] + b[...])

    x21 = _bn(gcn2(xp_ref[...], w21, b21).reshape(G * POOL, 30),
              g21[...], be21[...])
    x22 = _bn(gcn2(x21.reshape(G, POOL, 30), w22, b22).reshape(G * POOL, 30),
              g22[...], be22[...])
    x23 = _bn(gcn2(x22.reshape(G, POOL, 30), w23, b23).reshape(G * POOL, 30),
              g23[...], be23[...])
    x2 = jnp.concatenate([x21, x22, x23], axis=-1)
    x2_out = x2.reshape(G, POOL, 90).max(axis=1)
    x1_out = x1c_ref[...].reshape(G, NPG, 90).max(axis=1)
    conv_out = jnp.concatenate([x1_out, x2_out], axis=-1)
    hmid = jnp.maximum(
        jnp.dot(conv_out, wf1[...], preferred_element_type=jnp.float32)
        + bf1[...], 0.0)
    out_ref[...] = (jnp.dot(hmid, wf2[...],
                            preferred_element_type=jnp.float32) + bf2[...])


def _pooled(tparts_list, s_flat, xp, x1c, p):
    nt = len(tparts_list)

    def body(*refs):
        tp_refs = refs[:2 * nt]
        s_ref, xp_ref, x1c_ref = refs[2 * nt:2 * nt + 3]
        prm_refs = refs[2 * nt + 3:-1]
        _pooled_body(tp_refs, s_ref, xp_ref, x1c_ref, prm_refs, refs[-1])

    args = []
    for tp in tparts_list:
        args += [tp[0], tp[1]]
    args += [s_flat, xp, x1c]
    args += [p['W21'], p['b21'][None, :], p['g21'][None, :], p['be21'][None, :],
             p['W22'], p['b22'][None, :], p['g22'][None, :], p['be22'][None, :],
             p['W23'], p['b23'][None, :], p['g23'][None, :], p['be23'][None, :],
             p['Wf1'], p['bf1'][None, :], p['Wf2'], p['bf2'][None, :]]
    return pl.pallas_call(
        body,
        out_shape=jax.ShapeDtypeStruct((G, 6), jnp.float32),
    )(*args)


def kernel(x, edge_index, edge_attr, params):
    p = params
    src, dst = edge_index[0], edge_index[1]
    w = edge_attr

    src2d, dst2d = _pad_edges(src, dst)
    npad = EP - E
    coef_pad = lambda v: jnp.concatenate(
        [v, jnp.zeros((npad,), jnp.float32)]).reshape(EP // EB, EB)

    w_padded = jnp.concatenate([w, jnp.zeros((npad,), jnp.float32)])
    w2d = jnp.broadcast_to(w_padded[:, None], (EP, 32)) + jnp.zeros(
        (EP, 32), jnp.float32)

    ones_pad = jnp.zeros((NP, 32), jnp.float32).at[:N, :].set(1.0)
    dparts = _edge_segsum(ones_pad, src2d, dst2d, w2d)
    dinv = _dinv_kernel(dparts)

    def gcn1(xin, W, b, g, be):
        h, h_pad = _gcn_pre(xin, W, dinv)
        parts = _edge_segsum(h_pad, src2d, dst2d, w2d)
        return _gcn_post(parts, h, dinv, b, g, be)

    x11 = gcn1(x, p['W11'], p['b11'], p['g11'], p['be11'])
    x12 = gcn1(x11, p['W12'], p['b12'], p['g12'], p['be12'])
    x13 = gcn1(x12, p['W13'], p['b13'], p['g13'], p['be13'])
    x1c = jnp.concatenate([x11, x12, x13], axis=-1)

    psis = [jax.nn.softmax(
        jax.random.normal(jax.random.key(100 + i), (N, q)), axis=-1)
        for i, q in enumerate(QS)]
    psi0 = jnp.concatenate(psis, axis=-1)
    psi_pad = jnp.zeros((NP, 128), jnp.float32).at[:N, :QTOT].set(psi0)
    beta_row = jnp.zeros((1, 128), jnp.float32)
    for i in range(len(QS)):
        beta_row = beta_row.at[0, QOFF[i]:QOFF[i + 1]].set(p['beta'][i])
    for _ in range(10):
        parts01 = _edge_segsum(psi_pad, src2d, dst2d)
        psi_pad = _bp_softmax(parts01[0], parts01[1], beta_row)

    s_flat, xp = _assign(psi_pad, p['Wp'], p['bpb'], x13)

    seg2 = (src // NPG) * NPG + (dst % NPG)
    seg2d = jnp.concatenate(
        [seg2, jnp.full((EP - E,), N, jnp.int32)]).reshape(EP // EB, EB)
    w2d64 = jnp.broadcast_to(w_padded[:, None], (EP, 64)) + jnp.zeros(
        (EP, 64), jnp.float32)
    tparts_list = []
    for cb in range(0, POOL, 50):
        s_pad = jnp.zeros((NP, 64), jnp.float32).at[:N, :50].set(
            s_flat[:, cb:cb + 50])
        tparts_list.append(_edge_segsum(s_pad, src2d, seg2d, w2d64))

    out = _pooled(tparts_list, s_flat, xp, x1c, p)
    return (out, jnp.zeros((1,), jnp.float32))


# speedup vs baseline: 1.0773x; 1.0004x over previous
import functools
import numpy as np
import jax
import jax.numpy as jnp
from jax import lax
from jax.experimental import pallas as pl
from jax.experimental.pallas import tpu as pltpu, tpu_sc as plsc

N = 10000
G = 10
NPG = 1000
E = 160000
POOL = 100
QS = [2, 4, 8, 16, 32, 64]
QOFF = np.cumsum([0] + QS)
QTOT = int(QOFF[-1])

NP = 10240
EP = 163840
EB = 128
NBLK = EP // (32 * EB)
RPT = NP // 16


def _edge_segsum(table_pad, src2d, dst2d, coef2d=None):
    D = table_pad.shape[1]
    RING = 2 if D >= 128 else (4 if D == 64 else 6)
    HALF = RING // 2
    scaled = coef2d is not None
    mesh = plsc.VectorSubcoreMesh(core_axis_name="c", subcore_axis_name="s")

    scratch = [
        pltpu.VMEM((NBLK, EB), jnp.int32),
        pltpu.VMEM((NBLK, EB), jnp.int32),
    ]
    scratch += [pltpu.VMEM((EB, D), jnp.float32) for _ in range(RING)]
    if scaled:
        scratch += [pltpu.VMEM((EB, D), jnp.float32) for _ in range(RING)]
        scratch += [pltpu.SemaphoreType.DMA for _ in range(RING)]
    scratch += [pltpu.VMEM_SHARED((NP, D), jnp.float32)]
    scratch += [pltpu.SemaphoreType.DMA for _ in range(2 * RING)]

    @functools.partial(
        pl.kernel,
        out_type=jax.ShapeDtypeStruct((2, NP, D), jnp.float32),
        mesh=mesh,
        scratch_types=scratch,
        compiler_params=(None if D >= 128 else
                         pltpu.CompilerParams(use_tc_tiling_on_sc=False)),
    )
    def k(*refs):
        if scaled:
            (table_hbm, src_hbm, dst_hbm, cexp_hbm, zeros_hbm, out_hbm,
             srcv, dstv, *rest) = refs
            bufs = rest[:RING]
            cbufs = rest[RING:2 * RING]
            csems = rest[2 * RING:3 * RING]
            rest = rest[3 * RING:]
        else:
            (table_hbm, src_hbm, dst_hbm, zeros_hbm, out_hbm,
             srcv, dstv, *rest) = refs
            cexp_hbm = None
            bufs = rest[:RING]
            rest = rest[RING:]
        accum = rest[0]
        gsems = rest[1:1 + RING]
        ssems = rest[1 + RING:]
        c = lax.axis_index("c")
        s = lax.axis_index("s")
        tile = c * 16 + s
        pltpu.sync_copy(src_hbm.at[pl.ds(tile * NBLK, NBLK)], srcv)
        pltpu.sync_copy(dst_hbm.at[pl.ds(tile * NBLK, NBLK)], dstv)
        pltpu.sync_copy(zeros_hbm.at[pl.ds(s * RPT, RPT)],
                        accum.at[pl.ds(s * RPT, RPT)])
        plsc.subcore_barrier()

        def scale_block(buf, cbuf):
            def body(e, _):
                for k2 in range(D // 16):
                    buf[e, pl.ds(k2 * 16, 16)] = (
                        buf[e, pl.ds(k2 * 16, 16)]
                        * cbuf[e, pl.ds(k2 * 16, 16)])
                return 0

            lax.fori_loop(0, EB, body, 0)

        cd = [None] * NBLK
        gd = [None] * NBLK
        def start_block(j):
            b = j % RING
            gd[j] = pltpu.async_copy(
                table_hbm.at[srcv.at[j]], bufs[b], gsems[b])
            if scaled:
                cd[j] = pltpu.async_copy(
                    cexp_hbm.at[pl.ds((tile * NBLK + j) * EB, EB)],
                    cbufs[b], csems[b])

        sd = [None] * NBLK
        for j in range(HALF):
            start_block(j)
        for j in range(NBLK):
            jn = j + HALF
            if jn < NBLK:
                if jn - RING >= 0:
                    sd[jn - RING].wait()
                start_block(jn)
            b = j % RING
            gd[j].wait()
            if scaled:
                cd[j].wait()
                scale_block(bufs[b], cbufs[b])
            sd[j] = pltpu.async_copy(
                bufs[b], accum.at[dstv.at[j]], ssems[b], add=True)
        for j in range(max(0, NBLK - RING + HALF - HALF), NBLK):
            if j >= NBLK - RING:
                sd[j].wait()
        plsc.subcore_barrier()
        pltpu.sync_copy(accum.at[pl.ds(s * RPT, RPT)],
                        out_hbm.at[c, pl.ds(s * RPT, RPT)])

    zeros_pad = jnp.zeros((NP, D), jnp.float32)
    if scaled:
        return k(table_pad, src2d, dst2d, coef2d, zeros_pad)
    return k(table_pad, src2d, dst2d, zeros_pad)


def _bp_softmax_body(p0_ref, p1_ref, beta_ref, out_ref):
    m = (p0_ref[...] + p1_ref[...]) * beta_ref[...]
    parts = []
    for i, q in enumerate(QS):
        sub = m[:, QOFF[i]:QOFF[i + 1]]
        mx = jnp.max(sub, axis=-1, keepdims=True)
        e = jnp.exp(sub - mx)
        parts.append(e / jnp.sum(e, axis=-1, keepdims=True))
    parts.append(jnp.zeros((m.shape[0], 128 - QTOT), jnp.float32))
    out_ref[...] = jnp.concatenate(parts, axis=-1)


def _bp_softmax(part0, part1, beta_row):
    blk = 1280
    return pl.pallas_call(
        _bp_softmax_body,
        grid=(NP // blk,),
        in_specs=[
            pl.BlockSpec((blk, 128), lambda i: (i, 0)),
            pl.BlockSpec((blk, 128), lambda i: (i, 0)),
            pl.BlockSpec((1, 128), lambda i: (0, 0)),
        ],
        out_specs=pl.BlockSpec((blk, 128), lambda i: (i, 0)),
        out_shape=jax.ShapeDtypeStruct((NP, 128), jnp.float32),
    )(part0, part1, beta_row)


def _pad_edges(src, dst):
    npad = EP - E
    src_p = jnp.concatenate([src, jnp.full((npad,), N, jnp.int32)])
    dst_p = jnp.concatenate([dst, jnp.full((npad,), N, jnp.int32)])
    return src_p.reshape(EP // EB, EB), dst_p.reshape(EP // EB, EB)


def _bn(x, g, b):
    m = x.mean(0)
    v = x.var(0)
    return (x - m) / jnp.sqrt(v + 1e-5) * g + b


def _dinv_body(p0_ref, p1_ref, out_ref):
    deg = p0_ref[:N, :1] + p1_ref[:N, :1] + 1.0
    out_ref[...] = 1.0 / jnp.sqrt(deg)


def _dinv_kernel(dparts):
    return pl.pallas_call(
        _dinv_body,
        out_shape=jax.ShapeDtypeStruct((N, 1), jnp.float32),
    )(dparts[0], dparts[1])


def _gcn_pre_body(xp_ref, w_ref, dinv_ref, h_ref, htp_ref):
    h = jnp.dot(xp_ref[...], w_ref[...], preferred_element_type=jnp.float32)
    h_ref[...] = h
    ht = h * dinv_ref[...]
    ht32 = jnp.concatenate([ht, jnp.zeros((N, 2), jnp.float32)], axis=1)
    htp_ref[...] = jnp.concatenate(
        [ht32, jnp.zeros((NP - N, 32), jnp.float32)], axis=0)


def _gcn_pre(xin, W, dinv):
    return pl.pallas_call(
        _gcn_pre_body,
        out_shape=(jax.ShapeDtypeStruct((N, 30), jnp.float32),
                   jax.ShapeDtypeStruct((NP, 32), jnp.float32)),
    )(xin, W, dinv)


def _gcn_post_body(p0_ref, p1_ref, h_ref, dinv_ref, b_ref, g_ref, be_ref,
                   out_ref):
    dinv = dinv_ref[...]
    agg = dinv * (p0_ref[:N, :30] + p1_ref[:N, :30])
    pre = agg + h_ref[...] * (dinv * dinv) + b_ref[...]
    out_ref[...] = _bn(pre, g_ref[...], be_ref[...])


def _gcn_post(parts, h, dinv, b, g, be):
    return pl.pallas_call(
        _gcn_post_body,
        out_shape=jax.ShapeDtypeStruct((N, 30), jnp.float32),
    )(parts[0], parts[1], h, dinv, b[None, :], g[None, :], be[None, :])


def _assign_body(psi_ref, wp_ref, bpb_ref, x13_ref, s_ref, xp_ref):
    s1 = jnp.dot(psi_ref[:N, :QTOT], wp_ref[...],
                 preferred_element_type=jnp.float32) + bpb_ref[...]
    mx = jnp.max(s1, axis=-1, keepdims=True)
    e = jnp.exp(s1 - mx)
    s_flat = e / jnp.sum(e, axis=-1, keepdims=True)
    s_ref[...] = s_flat
    s3 = s_flat.reshape(G, NPG, POOL)
    x13r = x13_ref[...].reshape(G, NPG, 30)
    xp_ref[...] = jnp.einsum('gnk,gnd->gkd', s3, x13r,
                             preferred_element_type=jnp.float32)


def _assign(psi_pad, Wp, bpb, x13):
    return pl.pallas_call(
        _assign_body,
        out_shape=(jax.ShapeDtypeStruct((N, POOL), jnp.float32),
                   jax.ShapeDtypeStruct((G, POOL, 30), jnp.float32)),
    )(psi_pad, Wp, bpb[None, :], x13)


def _pooled_body(tp_refs, s_ref, xp_ref, x1c_ref, prm_refs, out_ref):
    (w21, b21, g21, be21, w22, b22, g22, be22,
     w23, b23, g23, be23, wf1, bf1, wf2, bf2) = prm_refs
    t_cols = []
    for c in range(2):
        p0, p1 = tp_refs[2 * c], tp_refs[2 * c + 1]
        t_cols.append(p0[:N, :50] + p1[:N, :50])
    t_seg = jnp.concatenate(t_cols, axis=-1).reshape(G, NPG, POOL)
    s3 = s_ref[...].reshape(G, NPG, POOL)
    adjp = jnp.einsum('gmk,gml->gkl', t_seg, s3,
                      preferred_element_type=jnp.float32)
    deg2 = adjp.sum(axis=1) + 1.0
    dinv2 = 1.0 / jnp.sqrt(deg2)

    def gcn2(h, W, b):
        hh = jnp.einsum('gnd,de->gne', h, W[...],
                        preferred_element_type=jnp.float32)
        hs = hh * dinv2[:, :, None]
        agg = jnp.einsum('gij,gid->gjd', adjp, hs,
                         preferred_element_type=jnp.float32)
        return (agg * dinv2[:, :, None]
                + hh * (dinv2 ** 2)[:, :, None] + b[...])

    x21 = _bn(gcn2(xp_ref[...], w21, b21).reshape(G * POOL, 30),
              g21[...], be21[...])
    x22 = _bn(gcn2(x21.reshape(G, POOL, 30), w22, b22).reshape(G * POOL, 30),
              g22[...], be22[...])
    x23 = _bn(gcn2(x22.reshape(G, POOL, 30), w23, b23).reshape(G * POOL, 30),
              g23[...], be23[...])
    x2 = jnp.concatenate([x21, x22, x23], axis=-1)
    x2_out = x2.reshape(G, POOL, 90).max(axis=1)
    x1_out = x1c_ref[...].reshape(G, NPG, 90).max(axis=1)
    conv_out = jnp.concatenate([x1_out, x2_out], axis=-1)
    hmid = jnp.maximum(
        jnp.dot(conv_out, wf1[...], preferred_element_type=jnp.float32)
        + bf1[...], 0.0)
    out_ref[...] = (jnp.dot(hmid, wf2[...],
                            preferred_element_type=jnp.float32) + bf2[...])


def _pooled(tparts_list, s_flat, xp, x1c, p):
    nt = len(tparts_list)

    def body(*refs):
        tp_refs = refs[:2 * nt]
        s_ref, xp_ref, x1c_ref = refs[2 * nt:2 * nt + 3]
        prm_refs = refs[2 * nt + 3:-1]
        _pooled_body(tp_refs, s_ref, xp_ref, x1c_ref, prm_refs, refs[-1])

    args = []
    for tp in tparts_list:
        args += [tp[0], tp[1]]
    args += [s_flat, xp, x1c]
    args += [p['W21'], p['b21'][None, :], p['g21'][None, :], p['be21'][None, :],
             p['W22'], p['b22'][None, :], p['g22'][None, :], p['be22'][None, :],
             p['W23'], p['b23'][None, :], p['g23'][None, :], p['be23'][None, :],
             p['Wf1'], p['bf1'][None, :], p['Wf2'], p['bf2'][None, :]]
    return pl.pallas_call(
        body,
        out_shape=jax.ShapeDtypeStruct((G, 6), jnp.float32),
    )(*args)


def kernel(x, edge_index, edge_attr, params):
    p = params
    src, dst = edge_index[0], edge_index[1]
    w = edge_attr

    src2d, dst2d = _pad_edges(src, dst)
    npad = EP - E
    w_padded = jnp.concatenate([w, jnp.zeros((npad,), jnp.float32)])
    w2d = jnp.broadcast_to(w_padded[:, None], (EP, 32)) + jnp.zeros(
        (EP, 32), jnp.float32)

    ones_pad = jnp.zeros((NP, 32), jnp.float32).at[:N, :].set(1.0)
    dparts = _edge_segsum(ones_pad, src2d, dst2d, w2d)
    dinv = _dinv_kernel(dparts)

    def gcn1(xin, W, b, g, be):
        h, h_pad = _gcn_pre(xin, W, dinv)
        parts = _edge_segsum(h_pad, src2d, dst2d, w2d)
        return _gcn_post(parts, h, dinv, b, g, be)

    x11 = gcn1(x, p['W11'], p['b11'], p['g11'], p['be11'])
    x12 = gcn1(x11, p['W12'], p['b12'], p['g12'], p['be12'])
    x13 = gcn1(x12, p['W13'], p['b13'], p['g13'], p['be13'])
    x1c = jnp.concatenate([x11, x12, x13], axis=-1)

    psis = [jax.nn.softmax(
        jax.random.normal(jax.random.key(100 + i), (N, q)), axis=-1)
        for i, q in enumerate(QS)]
    psi0 = jnp.concatenate(psis, axis=-1)
    psi_pad = jnp.zeros((NP, 128), jnp.float32).at[:N, :QTOT].set(psi0)
    beta_row = jnp.zeros((1, 128), jnp.float32)
    for i in range(len(QS)):
        beta_row = beta_row.at[0, QOFF[i]:QOFF[i + 1]].set(p['beta'][i])
    for _ in range(10):
        parts01 = _edge_segsum(psi_pad, src2d, dst2d)
        psi_pad = _bp_softmax(parts01[0], parts01[1], beta_row)

    s_flat, xp = _assign(psi_pad, p['Wp'], p['bpb'], x13)

    seg2 = (src // NPG) * NPG + (dst % NPG)
    seg2d = jnp.concatenate(
        [seg2, jnp.full((EP - E,), N, jnp.int32)]).reshape(EP // EB, EB)
    w2d64 = jnp.broadcast_to(w_padded[:, None], (EP, 64)) + jnp.zeros(
        (EP, 64), jnp.float32)
    tparts_list = []
    for cb in range(0, POOL, 50):
        s_pad = jnp.zeros((NP, 64), jnp.float32).at[:N, :50].set(
            s_flat[:, cb:cb + 50])
        tparts_list.append(_edge_segsum(s_pad, src2d, seg2d, w2d64))

    out = _pooled(tparts_list, s_flat, xp, x1c, p)
    return (out, jnp.zeros((1,), jnp.float32))
